# Initial kernel scaffold; baseline (speedup 1.0000x reference)
import jax
import jax.numpy as jnp
from jax.experimental import pallas as pl


def kernel(seq1, seq2, adj, raw_adj, normal_prompt, abnormal_prompt, W1, b1, a1, W2, b2, a2, Wb, bb, Wfc2, Wnp, Wap):
    raise NotImplementedError("write your pallas kernel here")


# speedup vs baseline: 1.4575x; 1.4575x over previous
import jax
import jax.numpy as jnp
from jax.experimental import pallas as pl
from jax.experimental.pallas import tpu as pltpu

N = 4096
D = 256
H = 64
BLK = 256
FBLK = 1024


def _feat_kernel(s1_ref, s2_ref, w1_ref, np_ref, wnp_ref, ap_ref, wap_ref,
                 ft_ref, npo_ref, apo_ref):
    f1 = jnp.dot(s1_ref[...], w1_ref[...], preferred_element_type=jnp.float32)
    f2 = jnp.dot(s2_ref[...], w1_ref[...], preferred_element_type=jnp.float32)
    ft_ref[...] = jnp.concatenate([f1, f2], axis=1)

    @pl.when(pl.program_id(0) == 0)
    def _():
        npo_ref[...] = jnp.dot(np_ref[...], wnp_ref[...],
                               preferred_element_type=jnp.float32)
        apo_ref[...] = jnp.dot(ap_ref[...], wap_ref[...],
                               preferred_element_type=jnp.float32)


def _pass1_kernel(adj_ref, ft_ref, b1_ref, a1_ref, h_ref, hsum_ref):
    acc = jnp.dot(adj_ref[...], ft_ref[...],
                  preferred_element_type=jnp.float32) + b1_ref[...]
    a = a1_ref[0, 0]
    h = jnp.where(acc >= 0, acc, a * acc)
    h_ref[...] = h
    s = jnp.sum(h[:, :H], axis=0, keepdims=True)

    @pl.when(pl.program_id(0) == 0)
    def _():
        hsum_ref[...] = jnp.zeros_like(hsum_ref)
    hsum_ref[...] += s


def _pass2_kernel(adj_ref, h_ref, w2_ref, b2_ref, a2_ref, wb_ref, hsum_ref,
                  bb_ref, y_ref, sc1_ref, sc2_ref, x_ref, v_ref):
    i = pl.program_id(0)

    @pl.when(i == 0)
    def _():
        h = h_ref[...]
        x1 = jnp.dot(h[:, :H], w2_ref[...], preferred_element_type=jnp.float32)
        x2 = jnp.dot(h[:, H:], w2_ref[...], preferred_element_type=jnp.float32)
        x_ref[...] = jnp.concatenate([x1, x2], axis=1)
        c = jax.nn.sigmoid(hsum_ref[...] * (1.0 / N))
        v_ref[...] = jnp.dot(wb_ref[...], c.T,
                             preferred_element_type=jnp.float32)

    acc = jnp.dot(adj_ref[...], x_ref[...],
                  preferred_element_type=jnp.float32) + b2_ref[...]
    a = a2_ref[0, 0]
    o = jnp.where(acc >= 0, acc, a * acc)
    y_ref[...] = o

    base = i * BLK
    v = v_ref[...]
    bb = bb_ref[0, 0]
    sc1_ref[pl.ds(base, BLK), :] = jnp.dot(
        h_ref[pl.ds(base, BLK), :H], v, preferred_element_type=jnp.float32) + bb
    sc2_ref[pl.ds(base, BLK), :] = jnp.dot(
        o[:, H:], v, preferred_element_type=jnp.float32) + bb


def _pass3_kernel(raw_ref, y_ref, wfc2_ref, resid_ref, logit_ref):
    i = pl.program_id(0)
    base = i * BLK
    raw = raw_ref[...]
    rows = jax.lax.broadcasted_iota(jnp.int32, (BLK, N), 0)
    cols = jax.lax.broadcasted_iota(jnp.int32, (BLK, N), 1)
    diag_mask = cols == rows + base
    d = jnp.sum(jnp.where(diag_mask, raw, 0.0), axis=1, keepdims=True)
    rs = jnp.sum(raw, axis=1, keepdims=True) - d
    emb_full = y_ref[:, :H]
    acc = jnp.dot(raw, emb_full, preferred_element_type=jnp.float32)
    emb_rows = y_ref[pl.ds(base, BLK), :H]
    num = acc - d * emb_rows
    safe = jnp.where(rs == 0.0, 1.0, rs)
    sub = jnp.where(rs == 0.0, 0.0, num / safe)
    resid = emb_rows - sub
    resid_ref[...] = resid
    logit_ref[pl.ds(base, BLK), :] = jnp.dot(
        resid, wfc2_ref[...], preferred_element_type=jnp.float32)


def kernel(seq1, seq2, adj, raw_adj, normal_prompt, abnormal_prompt,
           W1, b1, a1, W2, b2, a2, Wb, bb, Wfc2, Wnp, Wap):
    s1 = seq1.reshape(N, D)
    s2 = seq2.reshape(N, D)
    adj2 = adj.reshape(N, N)
    b1c = jnp.concatenate([b1, b1]).reshape(1, 2 * H)
    b2c = jnp.concatenate([b2, b2]).reshape(1, 2 * H)
    a1r = a1.reshape(1, 1)
    a2r = a2.reshape(1, 1)
    bbr = bb.reshape(1, 1)

    f32 = jnp.float32
    full = lambda shape: pl.BlockSpec(shape, lambda i: (0, 0))

    ft, np_out, ap_out = pl.pallas_call(
        _feat_kernel,
        grid=(N // FBLK,),
        in_specs=[
            pl.BlockSpec((FBLK, D), lambda i: (i, 0)),
            pl.BlockSpec((FBLK, D), lambda i: (i, 0)),
            full((D, H)), full((1, H)), full((H, H)), full((1, H)),
            full((H, H)),
        ],
        out_specs=[
            pl.BlockSpec((FBLK, 2 * H), lambda i: (i, 0)),
            full((1, H)), full((1, H)),
        ],
        out_shape=[
            jax.ShapeDtypeStruct((N, 2 * H), f32),
            jax.ShapeDtypeStruct((1, H), f32),
            jax.ShapeDtypeStruct((1, H), f32),
        ],
    )(s1, s2, W1, normal_prompt, Wnp, abnormal_prompt, Wap)

    h, hsum = pl.pallas_call(
        _pass1_kernel,
        grid=(N // BLK,),
        in_specs=[
            pl.BlockSpec((BLK, N), lambda i: (i, 0)),
            full((N, 2 * H)), full((1, 2 * H)), full((1, 1)),
        ],
        out_specs=[
            pl.BlockSpec((BLK, 2 * H), lambda i: (i, 0)),
            full((1, H)),
        ],
        out_shape=[
            jax.ShapeDtypeStruct((N, 2 * H), f32),
            jax.ShapeDtypeStruct((1, H), f32),
        ],
    )(adj2, ft, b1c, a1r)

    y, sc1, sc2 = pl.pallas_call(
        _pass2_kernel,
        grid=(N // BLK,),
        in_specs=[
            pl.BlockSpec((BLK, N), lambda i: (i, 0)),
            full((N, 2 * H)), full((H, H)), full((1, 2 * H)), full((1, 1)),
            full((H, H)), full((1, H)), full((1, 1)),
        ],
        out_specs=[
            pl.BlockSpec((BLK, 2 * H), lambda i: (i, 0)),
            full((N, 1)), full((N, 1)),
        ],
        out_shape=[
            jax.ShapeDtypeStruct((N, 2 * H), f32),
            jax.ShapeDtypeStruct((N, 1), f32),
            jax.ShapeDtypeStruct((N, 1), f32),
        ],
        scratch_shapes=[
            pltpu.VMEM((N, 2 * H), f32),
            pltpu.VMEM((H, 1), f32),
        ],
    )(adj2, h, W2, b2c, a2r, Wb, hsum, bbr)

    resid, logit = pl.pallas_call(
        _pass3_kernel,
        grid=(N // BLK,),
        in_specs=[
            pl.BlockSpec((BLK, N), lambda i: (i, 0)),
            full((N, 2 * H)), full((H, 1)),
        ],
        out_specs=[
            pl.BlockSpec((BLK, H), lambda i: (i, 0)),
            full((N, 1)),
        ],
        out_shape=[
            jax.ShapeDtypeStruct((N, H), f32),
            jax.ShapeDtypeStruct((N, 1), f32),
        ],
    )(raw_adj, y, Wfc2)

    ret = jnp.concatenate([sc1.reshape(1, N), sc2.reshape(1, N)], axis=1)
    emb = y[:, :H][None]
    return (ret, logit[None], emb, resid[None], np_out, ap_out)
